# baseline (speedup 1.0000x reference)
import dataclasses
import functools

import jax
import jax.numpy as jnp
from jax import lax
from jax.experimental import pallas as pl
from jax.experimental.pallas import tpu as pltpu
from jax.experimental.pallas import tpu_sc as plsc

N = 50000
R = 4
MAX_DELAY = 5
E = 1600000
B = 1
DT = 1.0
NZ = N * MAX_DELAY
W = 8192
NBITS = 32
NPAD = 50176
NC = 2
NS = 16
NW = NC * NS
EPT = E // NW
CHUNK = 10000
NCHUNK = EPT // CHUNK
PT = NPAD // NW
UNR = 5
PUNR = 1


def _pack_body(z_ref, out_ref):
    b = pl.program_id(0)
    term = lax.shift_left(z_ref[...].astype(jnp.int32), b)

    @pl.when(b == 0)
    def _init():
        out_ref[...] = term

    @pl.when(b > 0)
    def _acc():
        out_ref[...] += term


def _sc_edge_kernel(ept, e0, do_psc,
                    packed_hbm, rows_hbm, cols_hbm, w_hbm, pi_hbm, psc_hbm,
                    sd_hbm, out_hbm,
                    packed_v, acc_v, cols_a, cols_b, rows_a, rows_b, w_a, w_b,
                    pi_v, psc_v, sd_v, sem0, sem1):
    nchunk = ept // CHUNK
    wid = lax.axis_index("s") * NC + lax.axis_index("c")
    pltpu.sync_copy(packed_hbm.at[0], packed_v)
    pltpu.sync_copy(pi_hbm, pi_v)
    pltpu.sync_copy(sd_hbm, sd_v)
    pltpu.sync_copy(psc_hbm.at[0, pl.ds(wid * 4 * PT, 4 * PT)], psc_v)

    zeros16 = jnp.zeros((16,), jnp.float32)

    @pl.loop(0, NPAD, step=16)
    def _zero(i):
        acc_v[pl.ds(i, 16)] = zeros16

    if do_psc:
        lane = lax.broadcasted_iota(jnp.int32, (16,), 0)
        lane4 = lane * 4
        sdr = [plsc.load_gather(sd_v, [jnp.full((16,), r2 + 1, jnp.int32)])
               for r2 in range(R)]
        nbase = wid * PT

        @pl.loop(0, PT, step=16 * PUNR)
        def _psc(j):
            for u in range(PUNR):
                b4 = 4 * (j + 16 * u)
                s = plsc.load_gather(psc_v, [lane4 + b4]) * sdr[0]
                for r2 in range(1, R):
                    s = s + plsc.load_gather(psc_v, [lane4 + (b4 + r2)]) * sdr[r2]
                acc_v[pl.ds(nbase + j + 16 * u, 16)] = s

    base = e0 + wid * ept

    bufs = ((cols_a, rows_a, w_a, sem0), (cols_b, rows_b, w_b, sem1))

    def _fire(ci, bi):
        off = base + ci * CHUNK
        cv, rv, wv, sem = bufs[bi]
        return [
            pltpu.async_copy(rows_hbm.at[pl.ds(off, CHUNK)], rv, sem),
            pltpu.async_copy(cols_hbm.at[pl.ds(off, CHUNK)], cv, sem),
            pltpu.async_copy(w_hbm.at[pl.ds(off, CHUNK)], wv, sem),
        ]

    def _compute(bi):
        cv, rv, wv, _ = bufs[bi]

        @pl.loop(0, CHUNK, step=16 * UNR)
        def _edges(i):
            for u in range(UNR):
                ii = i + 16 * u
                rr = rv[pl.ds(ii, 16)]
                c = cv[pl.ds(ii, 16)]
                wd = plsc.load_gather(packed_v, [lax.bitwise_and(c, W - 1)])
                live = lax.bitwise_and(
                    lax.shift_right_logical(wd, lax.shift_right_logical(c, 13)),
                    1) == 1
                pv = plsc.load_gather(pi_v, [lax.bitwise_and(rr, 3)])
                val = wv[pl.ds(ii, 16)] * pv
                plsc.addupdate_scatter(
                    acc_v, [lax.shift_right_logical(rr, 2)], val, mask=live)

    pending = {0: _fire(0, 0), 1: None}
    for ci in range(nchunk):
        bi = ci & 1
        for h in pending[bi]:
            h.wait()
        if ci + 1 < nchunk:
            pending[1 - bi] = _fire(ci + 1, 1 - bi)
        _compute(bi)

    pltpu.sync_copy(acc_v, out_hbm.at[wid])


def _update_body(parts_ref, parts1_ref, v_ref, r_ref, a1_ref, a2_ref, ext_ref,
                 z_ref, decay_ref, cf_ref, vth_ref, vrst_ref, el_ref,
                 tref_ref, kd_ref, aa_ref, out_ref):
    psum = jnp.sum(parts_ref[...], axis=0) + jnp.sum(parts1_ref[...], axis=0)
    irec = psum.reshape(1, NPAD)[:, :N]
    prev_z = z_ref[...]
    a1 = kd_ref[0] * a1_ref[...] + prev_z * aa_ref[0]
    a2 = kd_ref[1] * a2_ref[...] + prev_z * aa_ref[1]
    cur = irec + ext_ref[...] + a1 + a2
    new_v = decay_ref[...] * v_ref[...] + cf_ref[...] * cur
    new_v = jnp.where(prev_z > 0.5, vrst_ref[...], new_v)
    new_r = jnp.maximum(r_ref[...] + prev_z * tref_ref[...] - DT, 0.0)
    v_sc = (new_v - vth_ref[...]) / (vth_ref[...] - el_ref[...] + 1e-8)
    z = (v_sc > 0.0).astype(jnp.float32)
    out_ref[...] = jnp.where(new_r > 0.0, 0.0, z)


def kernel(z_buf, v, r, asc_1, asc_2, psc, rec_weights, ext_current,
           syn_decay, psc_initial, asc_amps, k_decay, decay, current_factor,
           v_th, v_reset, e_l, t_ref, rec_indices):
    ea = 2 * E // 5
    rows_a = rec_indices[:ea, 0]
    cols_a = rec_indices[:ea, 1]
    rows_b = rec_indices[ea:, 0]
    cols_b = rec_indices[ea:, 1]
    pi16 = jnp.pad(psc_initial, (0, 16 - R))
    sd16 = jnp.pad(syn_decay, (1, 16 - R - 1))
    psc_sc = jnp.pad(psc, ((0, 0), (0, R * NPAD - R * N)))
    zp = jnp.pad(z_buf, ((0, 0), (0, NBITS * W - NZ)))

    packed = pl.pallas_call(
        _pack_body,
        grid=(NBITS,),
        in_specs=[pl.BlockSpec((1, W), lambda b: (0, b))],
        out_specs=pl.BlockSpec((1, W), lambda b: (0, 0)),
        out_shape=jax.ShapeDtypeStruct((1, W), jnp.int32),
    )(zp)

    mesh = plsc.VectorSubcoreMesh(core_axis_name="c", subcore_axis_name="s")
    cp = pltpu.CompilerParams()
    if "needs_layout_passes" in pltpu.CompilerParams.__dataclass_fields__:
        cp = dataclasses.replace(cp, needs_layout_passes=False)
    scratches = [
        pltpu.VMEM((W,), jnp.int32),
        pltpu.VMEM((NPAD,), jnp.float32),
        pltpu.VMEM((CHUNK,), jnp.int32),
        pltpu.VMEM((CHUNK,), jnp.int32),
        pltpu.VMEM((CHUNK,), jnp.int32),
        pltpu.VMEM((CHUNK,), jnp.int32),
        pltpu.VMEM((CHUNK,), jnp.float32),
        pltpu.VMEM((CHUNK,), jnp.float32),
        pltpu.VMEM((16,), jnp.float32),
        pltpu.VMEM((4 * PT,), jnp.float32),
        pltpu.VMEM((16,), jnp.float32),
        pltpu.SemaphoreType.DMA,
        pltpu.SemaphoreType.DMA,
    ]
    sc_edge_a = pl.kernel(
        functools.partial(_sc_edge_kernel, ea // NW, 0, True),
        out_type=jax.ShapeDtypeStruct((NW, NPAD), jnp.float32),
        mesh=mesh,
        scratch_types=scratches,
        compiler_params=cp,
    )
    sc_edge_b = pl.kernel(
        functools.partial(_sc_edge_kernel, (E - ea) // NW, 0, False),
        out_type=jax.ShapeDtypeStruct((NW, NPAD), jnp.float32),
        mesh=mesh,
        scratch_types=scratches,
        compiler_params=cp,
    )
    parts = sc_edge_a(packed, rows_a, cols_a, rec_weights[:ea], pi16,
                      psc_sc, sd16)
    parts1 = sc_edge_b(packed, rows_b, cols_b, rec_weights[ea:], pi16,
                       psc_sc, sd16)

    prev_z = z_buf[:, :N]
    smem_spec = pl.BlockSpec(memory_space=pltpu.SMEM)
    new_z = pl.pallas_call(
        _update_body,
        out_shape=jax.ShapeDtypeStruct((B, N), jnp.float32),
        in_specs=[pl.BlockSpec(memory_space=pltpu.VMEM)] * 14
        + [smem_spec, smem_spec],
        out_specs=pl.BlockSpec(memory_space=pltpu.VMEM),
    )(parts, parts1, v, r, asc_1, asc_2, ext_current, prev_z,
      decay.reshape(1, N), current_factor.reshape(1, N),
      v_th.reshape(1, N), v_reset.reshape(1, N), e_l.reshape(1, N),
      t_ref.reshape(1, N), k_decay, asc_amps)
    return new_z


# speedup vs baseline: 1.0882x; 1.0882x over previous
import dataclasses
import functools

import jax
import jax.numpy as jnp
from jax import lax
from jax.experimental import pallas as pl
from jax.experimental.pallas import tpu as pltpu
from jax.experimental.pallas import tpu_sc as plsc

N = 50000
R = 4
MAX_DELAY = 5
E = 1600000
B = 1
DT = 1.0
NZ = N * MAX_DELAY
W = 8192
NBITS = 32
NPAD = 50176
NC = 2
NS = 16
NW = NC * NS
EPT = E // NW
CHUNK = 10000
NCHUNK = EPT // CHUNK
PT = NPAD // NW
UNR = 5
PUNR = 1


def _pack_body(z_ref, out_ref):
    b = pl.program_id(0)
    term = lax.shift_left(z_ref[...].astype(jnp.int32), b)

    @pl.when(b == 0)
    def _init():
        out_ref[...] = term

    @pl.when(b > 0)
    def _acc():
        out_ref[...] += term


def _sc_edge_kernel(ept, e0, do_psc,
                    packed_hbm, rows_hbm, cols_hbm, w_hbm, pi_hbm, psc_hbm,
                    sd_hbm, out_hbm,
                    packed_v, acc_v, cols_a, cols_b, rows_a, rows_b, w_a, w_b,
                    pi_v, psc_v, sd_v, sem0, sem1):
    nchunk = ept // CHUNK
    wid = lax.axis_index("s") * NC + lax.axis_index("c")
    pltpu.sync_copy(packed_hbm.at[0], packed_v)
    pltpu.sync_copy(pi_hbm, pi_v)
    pltpu.sync_copy(sd_hbm, sd_v)
    pltpu.sync_copy(psc_hbm.at[0, pl.ds(wid * 4 * PT, 4 * PT)], psc_v)

    zeros16 = jnp.zeros((16,), jnp.float32)

    @pl.loop(0, NPAD, step=16)
    def _zero(i):
        acc_v[pl.ds(i, 16)] = zeros16

    if do_psc:
        lane = lax.broadcasted_iota(jnp.int32, (16,), 0)
        lane4 = lane * 4
        sdr = [plsc.load_gather(sd_v, [jnp.full((16,), r2 + 1, jnp.int32)])
               for r2 in range(R)]
        nbase = wid * PT

        @pl.loop(0, PT, step=16 * PUNR)
        def _psc(j):
            for u in range(PUNR):
                b4 = 4 * (j + 16 * u)
                s = plsc.load_gather(psc_v, [lane4 + b4]) * sdr[0]
                for r2 in range(1, R):
                    s = s + plsc.load_gather(psc_v, [lane4 + (b4 + r2)]) * sdr[r2]
                acc_v[pl.ds(nbase + j + 16 * u, 16)] = s

    base = e0 + wid * ept

    bufs = ((cols_a, rows_a, w_a, sem0), (cols_b, rows_b, w_b, sem1))

    def _fire(ci, bi):
        off = base + ci * CHUNK
        cv, rv, wv, sem = bufs[bi]
        return [
            pltpu.async_copy(rows_hbm.at[pl.ds(off, CHUNK)], rv, sem),
            pltpu.async_copy(cols_hbm.at[pl.ds(off, CHUNK)], cv, sem),
            pltpu.async_copy(w_hbm.at[pl.ds(off, CHUNK)], wv, sem),
        ]

    def _compute(bi):
        cv, rv, wv, _ = bufs[bi]

        @pl.loop(0, CHUNK, step=16 * UNR)
        def _edges(i):
            for u in range(UNR):
                ii = i + 16 * u
                rr = rv[pl.ds(ii, 16)]
                c = cv[pl.ds(ii, 16)]
                wd = plsc.load_gather(packed_v, [lax.bitwise_and(c, W - 1)])
                live = lax.bitwise_and(
                    lax.shift_right_logical(wd, lax.shift_right_logical(c, 13)),
                    1) == 1
                pv = plsc.load_gather(pi_v, [lax.bitwise_and(rr, 3)])
                val = wv[pl.ds(ii, 16)] * pv
                plsc.addupdate_scatter(
                    acc_v, [lax.shift_right_logical(rr, 2)], val, mask=live)

    pending = {0: _fire(0, 0), 1: None}
    for ci in range(nchunk):
        bi = ci & 1
        for h in pending[bi]:
            h.wait()
        if ci + 1 < nchunk:
            pending[1 - bi] = _fire(ci + 1, 1 - bi)
        _compute(bi)

    pltpu.sync_copy(acc_v, out_hbm.at[wid])


def _update_body(parts_ref, v_ref, r_ref, a1_ref, a2_ref, ext_ref,
                 z_ref, decay_ref, cf_ref, vth_ref, vrst_ref, el_ref,
                 tref_ref, kd_ref, aa_ref, out_ref):
    irec = jnp.sum(parts_ref[...], axis=0).reshape(1, NPAD)[:, :N]
    prev_z = z_ref[...]
    a1 = kd_ref[0] * a1_ref[...] + prev_z * aa_ref[0]
    a2 = kd_ref[1] * a2_ref[...] + prev_z * aa_ref[1]
    cur = irec + ext_ref[...] + a1 + a2
    new_v = decay_ref[...] * v_ref[...] + cf_ref[...] * cur
    new_v = jnp.where(prev_z > 0.5, vrst_ref[...], new_v)
    new_r = jnp.maximum(r_ref[...] + prev_z * tref_ref[...] - DT, 0.0)
    v_sc = (new_v - vth_ref[...]) / (vth_ref[...] - el_ref[...] + 1e-8)
    z = (v_sc > 0.0).astype(jnp.float32)
    out_ref[...] = jnp.where(new_r > 0.0, 0.0, z)


def kernel(z_buf, v, r, asc_1, asc_2, psc, rec_weights, ext_current,
           syn_decay, psc_initial, asc_amps, k_decay, decay, current_factor,
           v_th, v_reset, e_l, t_ref, rec_indices):
    rows = rec_indices[:, 0]
    cols = rec_indices[:, 1]
    pi16 = jnp.pad(psc_initial, (0, 16 - R))
    sd16 = jnp.pad(syn_decay, (1, 16 - R - 1))
    psc_sc = jnp.pad(psc, ((0, 0), (0, R * NPAD - R * N)))
    zp = jnp.pad(z_buf, ((0, 0), (0, NBITS * W - NZ)))

    packed = pl.pallas_call(
        _pack_body,
        grid=(NBITS,),
        in_specs=[pl.BlockSpec((1, W), lambda b: (0, b))],
        out_specs=pl.BlockSpec((1, W), lambda b: (0, 0)),
        out_shape=jax.ShapeDtypeStruct((1, W), jnp.int32),
    )(zp)

    mesh = plsc.VectorSubcoreMesh(core_axis_name="c", subcore_axis_name="s")
    cp = pltpu.CompilerParams()
    if "needs_layout_passes" in pltpu.CompilerParams.__dataclass_fields__:
        cp = dataclasses.replace(cp, needs_layout_passes=False)
    scratches = [
        pltpu.VMEM((W,), jnp.int32),
        pltpu.VMEM((NPAD,), jnp.float32),
        pltpu.VMEM((CHUNK,), jnp.int32),
        pltpu.VMEM((CHUNK,), jnp.int32),
        pltpu.VMEM((CHUNK,), jnp.int32),
        pltpu.VMEM((CHUNK,), jnp.int32),
        pltpu.VMEM((CHUNK,), jnp.float32),
        pltpu.VMEM((CHUNK,), jnp.float32),
        pltpu.VMEM((16,), jnp.float32),
        pltpu.VMEM((4 * PT,), jnp.float32),
        pltpu.VMEM((16,), jnp.float32),
        pltpu.SemaphoreType.DMA,
        pltpu.SemaphoreType.DMA,
    ]
    sc_edge = pl.kernel(
        functools.partial(_sc_edge_kernel, EPT, 0, True),
        out_type=jax.ShapeDtypeStruct((NW, NPAD), jnp.float32),
        mesh=mesh,
        scratch_types=scratches,
        compiler_params=cp,
    )
    parts = sc_edge(packed, rows, cols, rec_weights, pi16, psc_sc, sd16)

    prev_z = z_buf[:, :N]
    smem_spec = pl.BlockSpec(memory_space=pltpu.SMEM)
    new_z = pl.pallas_call(
        _update_body,
        out_shape=jax.ShapeDtypeStruct((B, N), jnp.float32),
        in_specs=[pl.BlockSpec(memory_space=pltpu.VMEM)] * 13
        + [smem_spec, smem_spec],
        out_specs=pl.BlockSpec(memory_space=pltpu.VMEM),
    )(parts, v, r, asc_1, asc_2, ext_current, prev_z,
      decay.reshape(1, N), current_factor.reshape(1, N),
      v_th.reshape(1, N), v_reset.reshape(1, N), e_l.reshape(1, N),
      t_ref.reshape(1, N), k_decay, asc_amps)
    return new_z
